# Initial kernel scaffold; baseline (speedup 1.0000x reference)
import jax
import jax.numpy as jnp
from jax.experimental import pallas as pl


def kernel(pc, params):
    raise NotImplementedError("write your pallas kernel here")


# speedup vs baseline: 7.0609x; 7.0609x over previous
import jax
import jax.numpy as jnp
from jax.experimental import pallas as pl

F32 = jnp.float32
I32 = jnp.int32
HI = jax.lax.Precision.HIGHEST

B = 32
N = 8192
S = 32
NS1 = 64
R2_1 = 0.02 ** 2
R2_2 = 0.04 ** 2


def _fps_kernel(xs_ref, ys_ref, zs_ref, cx_ref, cy_ref, cz_ref):
    xs = xs_ref[...]
    ys = ys_ref[...]
    zs = zs_ref[...]
    iota_n = jax.lax.broadcasted_iota(I32, (B, N), 1)
    iota_s = jax.lax.broadcasted_iota(I32, (B, S), 1)

    def body(t, carry):
        dists, far, ax, ay, az = carry
        sel = iota_n == far
        cx = jnp.sum(jnp.where(sel, xs, 0.0), axis=1, keepdims=True)
        cy = jnp.sum(jnp.where(sel, ys, 0.0), axis=1, keepdims=True)
        cz = jnp.sum(jnp.where(sel, zs, 0.0), axis=1, keepdims=True)
        ax = jnp.where(iota_s == t, cx, ax)
        ay = jnp.where(iota_s == t, cy, ay)
        az = jnp.where(iota_s == t, cz, az)
        dx = xs - cx
        dy = ys - cy
        dz = zs - cz
        d = dx * dx + dy * dy + dz * dz
        dists = jnp.minimum(dists, d)
        m = jnp.max(dists, axis=1, keepdims=True)
        far = jnp.min(jnp.where(dists == m, iota_n, N), axis=1, keepdims=True)
        return dists, far, ax, ay, az

    dists0 = jnp.full((B, N), 1e10, F32)
    far0 = jnp.zeros((B, 1), I32)
    a0 = jnp.zeros((B, S), F32)
    _, _, ax, ay, az = jax.lax.fori_loop(0, S, body, (dists0, far0, a0, a0, a0))
    cx_ref[...] = ax
    cy_ref[...] = ay
    cz_ref[...] = az


def _sa1_kernel(xt_ref, cxs_ref, cys_ref, czs_ref,
                w1_ref, b1_ref, g1_ref, e1_ref,
                w2_ref, b2_ref, g2_ref, e2_ref,
                w3_ref, b3_ref, g3_ref, e3_ref,
                o_ref):
    xt = xt_ref[0]
    xs = xt[0:1, :]
    ys = xt[1:2, :]
    zs = xt[2:3, :]
    cx = cxs_ref[0]
    cy = cys_ref[0]
    cz = czs_ref[0]
    dx = cx - xs
    dy = cy - ys
    dz = cz - zs
    d = dx * dx + dy * dy + dz * dz
    mask = d <= R2_1
    iota_n = jax.lax.broadcasted_iota(I32, (S, N), 1)
    cnt = jnp.sum(jnp.where(mask, 1.0, 0.0), axis=1, keepdims=True)
    lo = jnp.zeros((S, 1), I32)
    hi = jnp.full((S, 1), N, I32)
    for _ in range(13):
        mid = (lo + hi) // 2
        c = jnp.sum(jnp.where(mask & (iota_n < mid), 1.0, 0.0), axis=1,
                    keepdims=True)
        ge = c >= float(NS1)
        hi = jnp.where(ge, mid, hi)
        lo = jnp.where(ge, lo, mid)
    dmin = jnp.min(d, axis=1, keepdims=True)
    near = jnp.min(jnp.where(d == dmin, iota_n, N), axis=1, keepdims=True)
    inradf = jnp.where(mask & (iota_n < hi), 1.0, 0.0).astype(F32)
    nearf = jnp.where(iota_n == near, 1.0, 0.0).astype(F32)
    validf = jnp.where(cnt > 0, inradf, nearf)

    h = jnp.dot(w1_ref[...], xt, precision=HI) + b1_ref[...]
    h = jax.nn.relu(g1_ref[...] * h + e1_ref[...])
    h = jnp.dot(w2_ref[...], h, precision=HI) + b2_ref[...]
    h = jax.nn.relu(g2_ref[...] * h + e2_ref[...])
    h = jnp.dot(w3_ref[...], h, precision=HI) + b3_ref[...]
    f = jax.nn.relu(g3_ref[...] * h + e3_ref[...])

    lane_s = jax.lax.broadcasted_iota(I32, (128, S), 1)
    row_s = jax.lax.broadcasted_iota(I32, (S, N), 0)

    def pbody(s, acc):
        v = jnp.sum(jnp.where(row_s == s, validf, 0.0), axis=0,
                    keepdims=True)
        col = jnp.max(jnp.where(v > 0, f, 0.0), axis=1, keepdims=True)
        return jnp.where(lane_s == s, col, acc)

    acc = jax.lax.fori_loop(0, S, pbody, jnp.zeros((128, S), F32))
    o_ref[0] = acc


def _tail_kernel(cxl_ref, cyl_ref, czl_ref, cxs_ref, cys_ref, czs_ref,
                 f1_ref, *p_refs):
    p = [r[...] for r in p_refs[:-1]]
    o_ref = p_refs[-1]
    (w21, b21, g21, e21, w22, b22, g22, e22, w23, b23, g23, e23,
     w31, b31, g31, e31, w32, b32, g32, e32, w33, b33, g33, e33,
     wf1, bf1, gf1, ef1, wf2, bf2, gf2, ef2) = p

    xs = cxl_ref[...]
    ys = cyl_ref[...]
    zs = czl_ref[...]
    iota_n = jax.lax.broadcasted_iota(I32, (B, S), 1)

    def body(t, carry):
        dists, far, ax, ay, az = carry
        sel = iota_n == far
        cx = jnp.sum(jnp.where(sel, xs, 0.0), axis=1, keepdims=True)
        cy = jnp.sum(jnp.where(sel, ys, 0.0), axis=1, keepdims=True)
        cz = jnp.sum(jnp.where(sel, zs, 0.0), axis=1, keepdims=True)
        ax = jnp.where(iota_n == t, cx, ax)
        ay = jnp.where(iota_n == t, cy, ay)
        az = jnp.where(iota_n == t, cz, az)
        dx = xs - cx
        dy = ys - cy
        dz = zs - cz
        d = dx * dx + dy * dy + dz * dz
        dists = jnp.minimum(dists, d)
        m = jnp.max(dists, axis=1, keepdims=True)
        far = jnp.min(jnp.where(dists == m, iota_n, S), axis=1, keepdims=True)
        return dists, far, ax, ay, az

    dists0 = jnp.full((B, S), 1e10, F32)
    far0 = jnp.zeros((B, 1), I32)
    a0 = jnp.zeros((B, S), F32)
    _, _, c2x, c2y, c2z = jax.lax.fori_loop(0, S, body,
                                            (dists0, far0, a0, a0, a0))

    ddx = c2x.reshape(B, 1, S) - cxs_ref[...]
    ddy = c2y.reshape(B, 1, S) - cys_ref[...]
    ddz = c2z.reshape(B, 1, S) - czs_ref[...]
    d2 = ddx * ddx + ddy * ddy + ddz * ddz
    mask2 = d2 <= R2_2
    iota_3 = jax.lax.broadcasted_iota(I32, (B, S, S), 1)
    cnt2 = jnp.sum(jnp.where(mask2, 1.0, 0.0), axis=1, keepdims=True)
    dmin2 = jnp.min(d2, axis=1, keepdims=True)
    near2 = jnp.min(jnp.where(d2 == dmin2, iota_3, S), axis=1, keepdims=True)
    mask2f = jnp.where(mask2, 1.0, 0.0).astype(F32)
    near2f = jnp.where(iota_3 == near2, 1.0, 0.0).astype(F32)
    vf2 = jnp.where(cnt2 > 0, mask2f, near2f)

    x2 = f1_ref[...].reshape(B * S, 128)
    h = jnp.dot(x2, w21, precision=HI) + b21
    h = jax.nn.relu(g21 * h + e21)
    h = jnp.dot(h, w22, precision=HI) + b22
    h = jax.nn.relu(g22 * h + e22)
    h = jnp.dot(h, w23, precision=HI) + b23
    f2 = jax.nn.relu(g23 * h + e23)
    f2v = f2.reshape(B, S, 256)

    cols = []
    for s in range(S):
        m = vf2[:, :, s:s + 1]
        cols.append(jnp.max(jnp.where(m > 0, f2v, 0.0), axis=1, keepdims=True))
    out2 = jnp.concatenate(cols, axis=1)

    x3 = out2.reshape(B * S, 256)
    h = jnp.dot(x3, w31, precision=HI) + b31
    h = jax.nn.relu(g31 * h + e31)
    h = jnp.dot(h, w32, precision=HI) + b32
    h = jax.nn.relu(g32 * h + e32)
    h = jnp.dot(h, w33, precision=HI) + b33
    f3 = jax.nn.relu(g33 * h + e33)
    pooled = jnp.max(f3.reshape(B, S, 512), axis=1)

    y = jnp.dot(pooled, wf1, precision=HI) + bf1
    y = jax.nn.relu(gf1 * y + ef1)
    y = jnp.dot(y, wf2, precision=HI) + bf2
    y = jax.nn.relu(gf2 * y + ef2)
    o_ref[...] = y


def kernel(pc, params):
    pc = pc.astype(F32)
    xs = pc[:, :, 0]
    ys = pc[:, :, 1]
    zs = pc[:, :, 2]

    cx, cy, cz = pl.pallas_call(
        _fps_kernel,
        out_shape=[jax.ShapeDtypeStruct((B, S), F32)] * 3,
    )(xs, ys, zs)

    cxs = cx[:, :, None]
    cys = cy[:, :, None]
    czs = cz[:, :, None]
    xt = jnp.concatenate(
        [jnp.transpose(pc, (0, 2, 1)), jnp.zeros((B, 2, N), F32)], axis=1)

    sa1 = params["sa1"]
    w_in = []
    for (w, b, g, e) in sa1:
        wt = jnp.transpose(w)
        if wt.shape[1] == 6:
            wt = jnp.concatenate([wt, jnp.zeros((wt.shape[0], 2), F32)], axis=1)
        w_in += [wt, b[:, None], g[:, None], e[:, None]]

    grid = (B,)
    f1t = pl.pallas_call(
        _sa1_kernel,
        grid=grid,
        in_specs=[
            pl.BlockSpec((1, 8, N), lambda b: (b, 0, 0)),
            pl.BlockSpec((1, S, 1), lambda b: (b, 0, 0)),
            pl.BlockSpec((1, S, 1), lambda b: (b, 0, 0)),
            pl.BlockSpec((1, S, 1), lambda b: (b, 0, 0)),
        ] + [pl.BlockSpec(w.shape, lambda b: tuple(0 for _ in w.shape))
             for w in w_in],
        out_specs=pl.BlockSpec((1, 128, S), lambda b: (b, 0, 0)),
        out_shape=jax.ShapeDtypeStruct((B, 128, S), F32),
    )(xt, cxs, cys, czs, *w_in)

    f1 = jnp.transpose(f1t, (0, 2, 1))

    p_in = []
    for name in ("sa2", "sa3", "fc"):
        for (w, b, g, e) in params[name]:
            p_in += [w, b[None, :], g[None, :], e[None, :]]

    out = pl.pallas_call(
        _tail_kernel,
        out_shape=jax.ShapeDtypeStruct((B, 512), F32),
    )(cx, cy, cz, cxs, cys, czs, f1, *p_in)
    return out


# speedup vs baseline: 7.9103x; 1.1203x over previous
import jax
import jax.numpy as jnp
from jax.experimental import pallas as pl

F32 = jnp.float32
I32 = jnp.int32
HI = None

B = 32
N = 8192
S = 32
NS1 = 64
R2_1 = 0.02 ** 2
R2_2 = 0.04 ** 2


def _fps_kernel(xs_ref, ys_ref, zs_ref, cx_ref, cy_ref, cz_ref):
    xs = xs_ref[...]
    ys = ys_ref[...]
    zs = zs_ref[...]
    iota_n = jax.lax.broadcasted_iota(I32, (B, N), 1)
    iota_s = jax.lax.broadcasted_iota(I32, (B, S), 1)

    def body(t, carry):
        dists, far, ax, ay, az = carry
        sel = iota_n == far
        cx = jnp.sum(jnp.where(sel, xs, 0.0), axis=1, keepdims=True)
        cy = jnp.sum(jnp.where(sel, ys, 0.0), axis=1, keepdims=True)
        cz = jnp.sum(jnp.where(sel, zs, 0.0), axis=1, keepdims=True)
        ax = jnp.where(iota_s == t, cx, ax)
        ay = jnp.where(iota_s == t, cy, ay)
        az = jnp.where(iota_s == t, cz, az)
        dx = xs - cx
        dy = ys - cy
        dz = zs - cz
        d = dx * dx + dy * dy + dz * dz
        dists = jnp.minimum(dists, d)
        m = jnp.max(dists, axis=1, keepdims=True)
        far = jnp.min(jnp.where(dists == m, iota_n, N), axis=1, keepdims=True)
        return dists, far, ax, ay, az

    dists0 = jnp.full((B, N), 1e10, F32)
    far0 = jnp.zeros((B, 1), I32)
    a0 = jnp.zeros((B, S), F32)
    _, _, ax, ay, az = jax.lax.fori_loop(0, S, body, (dists0, far0, a0, a0, a0))
    cx_ref[...] = ax
    cy_ref[...] = ay
    cz_ref[...] = az


def _sa1_kernel(xt_ref, cxs_ref, cys_ref, czs_ref,
                w1_ref, b1_ref, g1_ref, e1_ref,
                w2_ref, b2_ref, g2_ref, e2_ref,
                w3_ref, b3_ref, g3_ref, e3_ref,
                o_ref):
    xt = xt_ref[0]
    xs = xt[0:1, :]
    ys = xt[1:2, :]
    zs = xt[2:3, :]
    cx = cxs_ref[0]
    cy = cys_ref[0]
    cz = czs_ref[0]
    dx = cx - xs
    dy = cy - ys
    dz = cz - zs
    d = dx * dx + dy * dy + dz * dz
    mask = d <= R2_1
    iota_n = jax.lax.broadcasted_iota(I32, (S, N), 1)
    cnt = jnp.sum(jnp.where(mask, 1.0, 0.0), axis=1, keepdims=True)
    lo = jnp.zeros((S, 1), I32)
    hi = jnp.full((S, 1), N, I32)
    for _ in range(13):
        mid = (lo + hi) // 2
        c = jnp.sum(jnp.where(mask & (iota_n < mid), 1.0, 0.0), axis=1,
                    keepdims=True)
        ge = c >= float(NS1)
        hi = jnp.where(ge, mid, hi)
        lo = jnp.where(ge, lo, mid)
    dmin = jnp.min(d, axis=1, keepdims=True)
    near = jnp.min(jnp.where(d == dmin, iota_n, N), axis=1, keepdims=True)
    inradf = jnp.where(mask & (iota_n < hi), 1.0, 0.0).astype(F32)
    nearf = jnp.where(iota_n == near, 1.0, 0.0).astype(F32)
    validf = jnp.where(cnt > 0, inradf, nearf)

    h = jnp.dot(w1_ref[...], xt, precision=HI) + b1_ref[...]
    h = jax.nn.relu(g1_ref[...] * h + e1_ref[...])
    h = jnp.dot(w2_ref[...], h, precision=HI) + b2_ref[...]
    h = jax.nn.relu(g2_ref[...] * h + e2_ref[...])
    h = jnp.dot(w3_ref[...], h, precision=HI) + b3_ref[...]
    f = jax.nn.relu(g3_ref[...] * h + e3_ref[...])

    lane_s = jax.lax.broadcasted_iota(I32, (128, S), 1)
    row_s = jax.lax.broadcasted_iota(I32, (S, N), 0)

    def pbody(s, acc):
        v = jnp.sum(jnp.where(row_s == s, validf, 0.0), axis=0,
                    keepdims=True)
        col = jnp.max(jnp.where(v > 0, f, 0.0), axis=1, keepdims=True)
        return jnp.where(lane_s == s, col, acc)

    acc = jax.lax.fori_loop(0, S, pbody, jnp.zeros((128, S), F32))
    o_ref[0] = acc


def _tail_kernel(cxl_ref, cyl_ref, czl_ref, cxs_ref, cys_ref, czs_ref,
                 f1_ref, *p_refs):
    p = [r[...] for r in p_refs[:-1]]
    o_ref = p_refs[-1]
    (w21, b21, g21, e21, w22, b22, g22, e22, w23, b23, g23, e23,
     w31, b31, g31, e31, w32, b32, g32, e32, w33, b33, g33, e33,
     wf1, bf1, gf1, ef1, wf2, bf2, gf2, ef2) = p

    xs = cxl_ref[...]
    ys = cyl_ref[...]
    zs = czl_ref[...]
    iota_n = jax.lax.broadcasted_iota(I32, (B, S), 1)

    def body(t, carry):
        dists, far, ax, ay, az = carry
        sel = iota_n == far
        cx = jnp.sum(jnp.where(sel, xs, 0.0), axis=1, keepdims=True)
        cy = jnp.sum(jnp.where(sel, ys, 0.0), axis=1, keepdims=True)
        cz = jnp.sum(jnp.where(sel, zs, 0.0), axis=1, keepdims=True)
        ax = jnp.where(iota_n == t, cx, ax)
        ay = jnp.where(iota_n == t, cy, ay)
        az = jnp.where(iota_n == t, cz, az)
        dx = xs - cx
        dy = ys - cy
        dz = zs - cz
        d = dx * dx + dy * dy + dz * dz
        dists = jnp.minimum(dists, d)
        m = jnp.max(dists, axis=1, keepdims=True)
        far = jnp.min(jnp.where(dists == m, iota_n, S), axis=1, keepdims=True)
        return dists, far, ax, ay, az

    dists0 = jnp.full((B, S), 1e10, F32)
    far0 = jnp.zeros((B, 1), I32)
    a0 = jnp.zeros((B, S), F32)
    _, _, c2x, c2y, c2z = jax.lax.fori_loop(0, S, body,
                                            (dists0, far0, a0, a0, a0))

    ddx = c2x.reshape(B, 1, S) - cxs_ref[...]
    ddy = c2y.reshape(B, 1, S) - cys_ref[...]
    ddz = c2z.reshape(B, 1, S) - czs_ref[...]
    d2 = ddx * ddx + ddy * ddy + ddz * ddz
    mask2 = d2 <= R2_2
    iota_3 = jax.lax.broadcasted_iota(I32, (B, S, S), 1)
    cnt2 = jnp.sum(jnp.where(mask2, 1.0, 0.0), axis=1, keepdims=True)
    dmin2 = jnp.min(d2, axis=1, keepdims=True)
    near2 = jnp.min(jnp.where(d2 == dmin2, iota_3, S), axis=1, keepdims=True)
    mask2f = jnp.where(mask2, 1.0, 0.0).astype(F32)
    near2f = jnp.where(iota_3 == near2, 1.0, 0.0).astype(F32)
    vf2 = jnp.where(cnt2 > 0, mask2f, near2f)

    x2 = f1_ref[...].reshape(B * S, 128)
    h = jnp.dot(x2, w21, precision=HI) + b21
    h = jax.nn.relu(g21 * h + e21)
    h = jnp.dot(h, w22, precision=HI) + b22
    h = jax.nn.relu(g22 * h + e22)
    h = jnp.dot(h, w23, precision=HI) + b23
    f2 = jax.nn.relu(g23 * h + e23)
    f2v = f2.reshape(B, S, 256)

    cols = []
    for s in range(S):
        m = vf2[:, :, s:s + 1]
        cols.append(jnp.max(jnp.where(m > 0, f2v, 0.0), axis=1, keepdims=True))
    out2 = jnp.concatenate(cols, axis=1)

    x3 = out2.reshape(B * S, 256)
    h = jnp.dot(x3, w31, precision=HI) + b31
    h = jax.nn.relu(g31 * h + e31)
    h = jnp.dot(h, w32, precision=HI) + b32
    h = jax.nn.relu(g32 * h + e32)
    h = jnp.dot(h, w33, precision=HI) + b33
    f3 = jax.nn.relu(g33 * h + e33)
    pooled = jnp.max(f3.reshape(B, S, 512), axis=1)

    y = jnp.dot(pooled, wf1, precision=HI) + bf1
    y = jax.nn.relu(gf1 * y + ef1)
    y = jnp.dot(y, wf2, precision=HI) + bf2
    y = jax.nn.relu(gf2 * y + ef2)
    o_ref[...] = y


def kernel(pc, params):
    pc = pc.astype(F32)
    xs = pc[:, :, 0]
    ys = pc[:, :, 1]
    zs = pc[:, :, 2]

    cx, cy, cz = pl.pallas_call(
        _fps_kernel,
        out_shape=[jax.ShapeDtypeStruct((B, S), F32)] * 3,
    )(xs, ys, zs)

    cxs = cx[:, :, None]
    cys = cy[:, :, None]
    czs = cz[:, :, None]
    xt = jnp.concatenate(
        [jnp.transpose(pc, (0, 2, 1)), jnp.zeros((B, 2, N), F32)], axis=1)

    sa1 = params["sa1"]
    w_in = []
    for (w, b, g, e) in sa1:
        wt = jnp.transpose(w)
        if wt.shape[1] == 6:
            wt = jnp.concatenate([wt, jnp.zeros((wt.shape[0], 2), F32)], axis=1)
        w_in += [wt, b[:, None], g[:, None], e[:, None]]

    grid = (B,)
    f1t = pl.pallas_call(
        _sa1_kernel,
        grid=grid,
        in_specs=[
            pl.BlockSpec((1, 8, N), lambda b: (b, 0, 0)),
            pl.BlockSpec((1, S, 1), lambda b: (b, 0, 0)),
            pl.BlockSpec((1, S, 1), lambda b: (b, 0, 0)),
            pl.BlockSpec((1, S, 1), lambda b: (b, 0, 0)),
        ] + [pl.BlockSpec(w.shape, lambda b: tuple(0 for _ in w.shape))
             for w in w_in],
        out_specs=pl.BlockSpec((1, 128, S), lambda b: (b, 0, 0)),
        out_shape=jax.ShapeDtypeStruct((B, 128, S), F32),
    )(xt, cxs, cys, czs, *w_in)

    f1 = jnp.transpose(f1t, (0, 2, 1))

    p_in = []
    for name in ("sa2", "sa3", "fc"):
        for (w, b, g, e) in params[name]:
            p_in += [w, b[None, :], g[None, :], e[None, :]]

    out = pl.pallas_call(
        _tail_kernel,
        out_shape=jax.ShapeDtypeStruct((B, 512), F32),
    )(cx, cy, cz, cxs, cys, czs, f1, *p_in)
    return out


# speedup vs baseline: 15.4590x; 1.9543x over previous
import functools

import jax
import jax.numpy as jnp
from jax import lax
from jax.experimental import pallas as pl
from jax.experimental.pallas import tpu as pltpu
from jax.experimental.pallas import tpu_sc as plsc

F32 = jnp.float32
I32 = jnp.int32
HI = None

B = 32
N = 8192
S = 32
NS1 = 64
NCHUNK = 64
R2_1 = 0.02 ** 2
R2_2 = 0.04 ** 2


def _fps_kernel(xs_ref, ys_ref, zs_ref, cx_ref, cy_ref, cz_ref):
    xs = xs_ref[...]
    ys = ys_ref[...]
    zs = zs_ref[...]
    iota_n = jax.lax.broadcasted_iota(I32, (B, N), 1)
    iota_s = jax.lax.broadcasted_iota(I32, (B, S), 1)

    def body(t, carry):
        dists, far, ax, ay, az = carry
        sel = iota_n == far
        cx = jnp.sum(jnp.where(sel, xs, 0.0), axis=1, keepdims=True)
        cy = jnp.sum(jnp.where(sel, ys, 0.0), axis=1, keepdims=True)
        cz = jnp.sum(jnp.where(sel, zs, 0.0), axis=1, keepdims=True)
        ax = jnp.where(iota_s == t, cx, ax)
        ay = jnp.where(iota_s == t, cy, ay)
        az = jnp.where(iota_s == t, cz, az)
        dx = xs - cx
        dy = ys - cy
        dz = zs - cz
        d = dx * dx + dy * dy + dz * dz
        dists = jnp.minimum(dists, d)
        m = jnp.max(dists, axis=1, keepdims=True)
        far = jnp.min(jnp.where(dists == m, iota_n, N), axis=1, keepdims=True)
        return dists, far, ax, ay, az

    dists0 = jnp.full((B, N), 1e10, F32)
    far0 = jnp.zeros((B, 1), I32)
    a0 = jnp.zeros((B, S), F32)
    _, _, ax, ay, az = jax.lax.fori_loop(0, S, body, (dists0, far0, a0, a0, a0))
    cx_ref[...] = ax
    cy_ref[...] = ay
    cz_ref[...] = az


def _mask_kernel(xt_ref, cxs_ref, cys_ref, czs_ref,
                 vmask_ref, ccnt_ref, near_ref):
    xt = xt_ref[0]
    xs = xt[0:1, :]
    ys = xt[1:2, :]
    zs = xt[2:3, :]
    cx = cxs_ref[0]
    cy = cys_ref[0]
    cz = czs_ref[0]
    dx = cx - xs
    dy = cy - ys
    dz = cz - zs
    d = dx * dx + dy * dy + dz * dz
    maskf = jnp.where(d <= R2_1, 1.0, 0.0).astype(F32)
    iota_n = jax.lax.broadcasted_iota(I32, (S, N), 1)
    dmin = jnp.min(d, axis=1, keepdims=True)
    near = jnp.min(jnp.where(d == dmin, iota_n, N), axis=1, keepdims=True)
    vmask_ref[0] = maskf
    csum = jnp.sum(maskf.reshape(S, NCHUNK, 128), axis=2)
    ccnt_ref[0] = csum.astype(I32)
    near_ref[0] = near


def _sc_gather_body(vmask_hbm, ccnt_hbm, near_hbm, table_hbm, out_hbm,
                    vrow, ccv, nearv, idxbuf, idxg, colp, sem):
    info = plsc.get_sparse_core_info()
    nc = info.num_cores
    b = lax.axis_index("s") * nc + lax.axis_index("c")
    iota16 = jax.lax.broadcasted_iota(I32, (16,), 0)

    pltpu.sync_copy(ccnt_hbm.at[pl.ds(b * S * NCHUNK, S * NCHUNK)],
                    ccv.at[pl.ds(0, S * NCHUNK)])
    pltpu.sync_copy(near_hbm.at[pl.ds(b * S, S)], nearv.at[pl.ds(0, S)])

    def s_body(s, _):
        pltpu.sync_copy(vmask_hbm.at[pl.ds((b * S + s) * N, N)], vrow)

        def c_body(c, off):
            cnt_c = ccv[pl.ds(s * NCHUNK + c, 16)][0]
            do = jnp.logical_and(cnt_c > 0, off < NS1)

            @pl.when(do)
            def _(off=off):
                sub = off
                for j in range(8):
                    v = vrow[pl.ds(c * 128 + j * 16, 16)]
                    for q in range(16):
                        pq = v[q] > 0.5

                        @pl.when(pq)
                        def _(sub=sub, j=j, q=q):
                            val = c * 128 + (j * 16 + q)
                            idxbuf[pl.ds(sub, 16)] = (
                                jnp.zeros((16,), I32) + val)

                        sub = sub + jnp.where(pq, 1, 0)

            return off + jnp.where(do, cnt_c, 0)

        off = jax.lax.fori_loop(0, NCHUNK, c_body, jnp.int32(0))

        first0 = idxbuf[pl.ds(0, 16)][0]
        near_s = nearv[pl.ds(s, 16)][0]
        first = jnp.where(off > 0, first0, near_s)
        ebase = b * N * 8
        for k in range(4):
            cur = idxbuf[pl.ds(k * 16, 16)]
            slot = iota16 + (k * 16)
            e_k = jnp.where(slot < off, cur, first) * 8 + ebase
            for j in range(8):
                idxg[pl.ds(j * (S * NS1) + s * NS1 + k * 16, 16)] = e_k + j
        return 0

    jax.lax.fori_loop(0, S, s_body, 0)

    waits = []
    for j in range(8):
        waits.append(pltpu.async_copy(
            table_hbm.at[idxg.at[pl.ds(j * (S * NS1), S * NS1)]],
            colp.at[pl.ds(j * (S * NS1), S * NS1)], sem))
    for w in waits:
        w.wait()
    pltpu.sync_copy(colp, out_hbm.at[pl.ds(b * 8 * S * NS1, 8 * S * NS1)])


def _mlp1_kernel(g_ref,
                 w1_ref, b1_ref, g1_ref, e1_ref,
                 w2_ref, b2_ref, g2_ref, e2_ref,
                 w3_ref, b3_ref, g3_ref, e3_ref,
                 o_ref):
    x = g_ref[0]
    h = jnp.dot(w1_ref[...], x, precision=HI) + b1_ref[...]
    h = jax.nn.relu(g1_ref[...] * h + e1_ref[...])
    h = jnp.dot(w2_ref[...], h, precision=HI) + b2_ref[...]
    h = jax.nn.relu(g2_ref[...] * h + e2_ref[...])
    h = jnp.dot(w3_ref[...], h, precision=HI) + b3_ref[...]
    f = jax.nn.relu(g3_ref[...] * h + e3_ref[...])
    o_ref[0] = jnp.max(f.reshape(128, S, NS1), axis=2)


def _tail_kernel(cxl_ref, cyl_ref, czl_ref, cxs_ref, cys_ref, czs_ref,
                 f1_ref, *p_refs):
    p = [r[...] for r in p_refs[:-1]]
    o_ref = p_refs[-1]
    (w21, b21, g21, e21, w22, b22, g22, e22, w23, b23, g23, e23,
     w31, b31, g31, e31, w32, b32, g32, e32, w33, b33, g33, e33,
     wf1, bf1, gf1, ef1, wf2, bf2, gf2, ef2) = p

    xs = cxl_ref[...]
    ys = cyl_ref[...]
    zs = czl_ref[...]
    iota_n = jax.lax.broadcasted_iota(I32, (B, S), 1)

    def body(t, carry):
        dists, far, ax, ay, az = carry
        sel = iota_n == far
        cx = jnp.sum(jnp.where(sel, xs, 0.0), axis=1, keepdims=True)
        cy = jnp.sum(jnp.where(sel, ys, 0.0), axis=1, keepdims=True)
        cz = jnp.sum(jnp.where(sel, zs, 0.0), axis=1, keepdims=True)
        ax = jnp.where(iota_n == t, cx, ax)
        ay = jnp.where(iota_n == t, cy, ay)
        az = jnp.where(iota_n == t, cz, az)
        dx = xs - cx
        dy = ys - cy
        dz = zs - cz
        d = dx * dx + dy * dy + dz * dz
        dists = jnp.minimum(dists, d)
        m = jnp.max(dists, axis=1, keepdims=True)
        far = jnp.min(jnp.where(dists == m, iota_n, S), axis=1, keepdims=True)
        return dists, far, ax, ay, az

    dists0 = jnp.full((B, S), 1e10, F32)
    far0 = jnp.zeros((B, 1), I32)
    a0 = jnp.zeros((B, S), F32)
    _, _, c2x, c2y, c2z = jax.lax.fori_loop(0, S, body,
                                            (dists0, far0, a0, a0, a0))

    ddx = c2x.reshape(B, 1, S) - cxs_ref[...]
    ddy = c2y.reshape(B, 1, S) - cys_ref[...]
    ddz = c2z.reshape(B, 1, S) - czs_ref[...]
    d2 = ddx * ddx + ddy * ddy + ddz * ddz
    mask2 = d2 <= R2_2
    iota_3 = jax.lax.broadcasted_iota(I32, (B, S, S), 1)
    cnt2 = jnp.sum(jnp.where(mask2, 1.0, 0.0), axis=1, keepdims=True)
    dmin2 = jnp.min(d2, axis=1, keepdims=True)
    near2 = jnp.min(jnp.where(d2 == dmin2, iota_3, S), axis=1, keepdims=True)
    mask2f = jnp.where(mask2, 1.0, 0.0).astype(F32)
    near2f = jnp.where(iota_3 == near2, 1.0, 0.0).astype(F32)
    vf2 = jnp.where(cnt2 > 0, mask2f, near2f)

    x2 = f1_ref[...].reshape(B * S, 128)
    h = jnp.dot(x2, w21, precision=HI) + b21
    h = jax.nn.relu(g21 * h + e21)
    h = jnp.dot(h, w22, precision=HI) + b22
    h = jax.nn.relu(g22 * h + e22)
    h = jnp.dot(h, w23, precision=HI) + b23
    f2 = jax.nn.relu(g23 * h + e23)
    f2v = f2.reshape(B, S, 256)

    cols = []
    for s in range(S):
        m = vf2[:, :, s:s + 1]
        cols.append(jnp.max(jnp.where(m > 0, f2v, 0.0), axis=1, keepdims=True))
    out2 = jnp.concatenate(cols, axis=1)

    x3 = out2.reshape(B * S, 256)
    h = jnp.dot(x3, w31, precision=HI) + b31
    h = jax.nn.relu(g31 * h + e31)
    h = jnp.dot(h, w32, precision=HI) + b32
    h = jax.nn.relu(g32 * h + e32)
    h = jnp.dot(h, w33, precision=HI) + b33
    f3 = jax.nn.relu(g33 * h + e33)
    pooled = jnp.max(f3.reshape(B, S, 512), axis=1)

    y = jnp.dot(pooled, wf1, precision=HI) + bf1
    y = jax.nn.relu(gf1 * y + ef1)
    y = jnp.dot(y, wf2, precision=HI) + bf2
    y = jax.nn.relu(gf2 * y + ef2)
    o_ref[...] = y


def kernel(pc, params):
    pc = pc.astype(F32)
    xs = pc[:, :, 0]
    ys = pc[:, :, 1]
    zs = pc[:, :, 2]

    cx, cy, cz = pl.pallas_call(
        _fps_kernel,
        out_shape=[jax.ShapeDtypeStruct((B, S), F32)] * 3,
    )(xs, ys, zs)

    cxs = cx[:, :, None]
    cys = cy[:, :, None]
    czs = cz[:, :, None]
    xt = jnp.concatenate(
        [jnp.transpose(pc, (0, 2, 1)), jnp.zeros((B, 2, N), F32)], axis=1)

    vmask, ccnt, near = pl.pallas_call(
        _mask_kernel,
        grid=(B,),
        in_specs=[
            pl.BlockSpec((1, 8, N), lambda b: (b, 0, 0)),
            pl.BlockSpec((1, S, 1), lambda b: (b, 0, 0)),
            pl.BlockSpec((1, S, 1), lambda b: (b, 0, 0)),
            pl.BlockSpec((1, S, 1), lambda b: (b, 0, 0)),
        ],
        out_specs=[
            pl.BlockSpec((1, S, N), lambda b: (b, 0, 0)),
            pl.BlockSpec((1, S, NCHUNK), lambda b: (b, 0, 0)),
            pl.BlockSpec((1, S, 1), lambda b: (b, 0, 0)),
        ],
        out_shape=[
            jax.ShapeDtypeStruct((B, S, N), F32),
            jax.ShapeDtypeStruct((B, S, NCHUNK), I32),
            jax.ShapeDtypeStruct((B, S, 1), I32),
        ],
    )(xt, cxs, cys, czs)

    table = jnp.concatenate([pc, jnp.zeros((B, N, 2), F32)],
                            axis=2).reshape(B * N * 8)

    mesh = plsc.VectorSubcoreMesh(core_axis_name="c", subcore_axis_name="s")
    grouped = pl.kernel(
        _sc_gather_body,
        mesh=mesh,
        out_type=jax.ShapeDtypeStruct((B * 8 * S * NS1,), F32),
        scratch_types=[
            pltpu.VMEM((N,), F32),
            pltpu.VMEM((S * NCHUNK + 16,), I32),
            pltpu.VMEM((S + 16,), I32),
            pltpu.VMEM((256,), I32),
            pltpu.VMEM((8 * S * NS1,), I32),
            pltpu.VMEM((8 * S * NS1,), F32),
            pltpu.SemaphoreType.DMA,
        ],
    )(vmask.reshape(B * S * N), ccnt.reshape(B * S * NCHUNK),
      near.reshape(B * S), table)

    sa1 = params["sa1"]
    w_in = []
    for (w, b, g, e) in sa1:
        wt = jnp.transpose(w)
        if wt.shape[1] == 6:
            wt = jnp.concatenate([wt, jnp.zeros((wt.shape[0], 2), F32)],
                                 axis=1)
        w_in += [wt, b[:, None], g[:, None], e[:, None]]

    f1t = pl.pallas_call(
        _mlp1_kernel,
        grid=(B,),
        in_specs=[
            pl.BlockSpec((1, 8, S * NS1), lambda b: (b, 0, 0)),
        ] + [pl.BlockSpec(w.shape, lambda b: (0, 0)) for w in w_in],
        out_specs=pl.BlockSpec((1, 128, S), lambda b: (b, 0, 0)),
        out_shape=jax.ShapeDtypeStruct((B, 128, S), F32),
    )(grouped.reshape(B, 8, S * NS1), *w_in)
    f1 = jnp.transpose(f1t, (0, 2, 1))

    p_in = []
    for name in ("sa2", "sa3", "fc"):
        for (w, b, g, e) in params[name]:
            p_in += [w, b[None, :], g[None, :], e[None, :]]

    out = pl.pallas_call(
        _tail_kernel,
        out_shape=jax.ShapeDtypeStruct((B, 512), F32),
    )(cx, cy, cz, cxs, cys, czs, f1, *p_in)
    return out


# speedup vs baseline: 15.4665x; 1.0005x over previous
import jax
import jax.numpy as jnp
from jax import lax
from jax.experimental import pallas as pl
from jax.experimental.pallas import tpu as pltpu
from jax.experimental.pallas import tpu_sc as plsc

F32 = jnp.float32
I32 = jnp.int32
HI = None

B = 32
N = 8192
S = 32
NS1 = 64
NCHUNK = 64
R2_1 = 0.02 ** 2
R2_2 = 0.04 ** 2


def _fps_kernel(xs_ref, ys_ref, zs_ref, cx_ref, cy_ref, cz_ref):
    xs = xs_ref[...]
    ys = ys_ref[...]
    zs = zs_ref[...]
    iota_n = jax.lax.broadcasted_iota(I32, (B, N), 1)
    iota_s = jax.lax.broadcasted_iota(I32, (B, S), 1)

    def body(t, carry):
        dists, far, ax, ay, az = carry
        sel = iota_n == far
        cx = jnp.sum(jnp.where(sel, xs, 0.0), axis=1, keepdims=True)
        cy = jnp.sum(jnp.where(sel, ys, 0.0), axis=1, keepdims=True)
        cz = jnp.sum(jnp.where(sel, zs, 0.0), axis=1, keepdims=True)
        ax = jnp.where(iota_s == t, cx, ax)
        ay = jnp.where(iota_s == t, cy, ay)
        az = jnp.where(iota_s == t, cz, az)
        dx = xs - cx
        dy = ys - cy
        dz = zs - cz
        d = dx * dx + dy * dy + dz * dz
        dists = jnp.minimum(dists, d)
        m = jnp.max(dists, axis=1, keepdims=True)
        far = jnp.min(jnp.where(dists == m, iota_n, N), axis=1, keepdims=True)
        return dists, far, ax, ay, az

    dists0 = jnp.full((B, N), 1e10, F32)
    far0 = jnp.zeros((B, 1), I32)
    a0 = jnp.zeros((B, S), F32)
    _, _, ax, ay, az = jax.lax.fori_loop(0, S, body, (dists0, far0, a0, a0, a0))
    cx_ref[...] = ax
    cy_ref[...] = ay
    cz_ref[...] = az


def _mask_kernel(xt_ref, cxs_ref, cys_ref, czs_ref,
                 vmask_ref, ccnt_ref, near_ref):
    xt = xt_ref[0]
    xs = xt[0:1, :]
    ys = xt[1:2, :]
    zs = xt[2:3, :]
    cx = cxs_ref[0]
    cy = cys_ref[0]
    cz = czs_ref[0]
    dx = cx - xs
    dy = cy - ys
    dz = cz - zs
    d = dx * dx + dy * dy + dz * dz
    maskf = jnp.where(d <= R2_1, 1.0, 0.0).astype(F32)
    iota_n = jax.lax.broadcasted_iota(I32, (S, N), 1)
    dmin = jnp.min(d, axis=1, keepdims=True)
    near = jnp.min(jnp.where(d == dmin, iota_n, N), axis=1, keepdims=True)
    vmask_ref[0] = maskf
    csum = jnp.sum(maskf.reshape(S, NCHUNK, 128), axis=2)
    ccnt_ref[0] = csum.astype(I32)
    near_ref[0] = near


def _sc_gather_body(vmask_hbm, ccnt_hbm, near_hbm, table_hbm, out_hbm,
                    vrow, ccv, nearv, idxbuf, idxg, colp, sem):
    info = plsc.get_sparse_core_info()
    nc = info.num_cores
    b = lax.axis_index("s") * nc + lax.axis_index("c")
    iota16 = jax.lax.broadcasted_iota(I32, (16,), 0)

    pltpu.sync_copy(ccnt_hbm.at[pl.ds(b * S * NCHUNK, S * NCHUNK)],
                    ccv.at[pl.ds(0, S * NCHUNK)])
    pltpu.sync_copy(near_hbm.at[pl.ds(b * S, S)], nearv.at[pl.ds(0, S)])

    def s_body(s, _):
        pltpu.sync_copy(vmask_hbm.at[pl.ds((b * S + s) * N, N)], vrow)

        def c_body(c, off):
            cnt_c = ccv[pl.ds(s * NCHUNK + c, 16)][0]
            do = jnp.logical_and(cnt_c > 0, off < NS1)

            @pl.when(do)
            def _(off=off):
                sub = off
                for j in range(8):
                    v = vrow[pl.ds(c * 128 + j * 16, 16)]
                    for q in range(16):
                        pq = v[q] > 0.5

                        @pl.when(pq)
                        def _(sub=sub, j=j, q=q):
                            val = c * 128 + (j * 16 + q)
                            idxbuf[pl.ds(sub, 16)] = (
                                jnp.zeros((16,), I32) + val)

                        sub = sub + jnp.where(pq, 1, 0)

            return off + jnp.where(do, cnt_c, 0)

        off = jax.lax.fori_loop(0, NCHUNK, c_body, jnp.int32(0))

        first0 = idxbuf[pl.ds(0, 16)][0]
        near_s = nearv[pl.ds(s, 16)][0]
        first = jnp.where(off > 0, first0, near_s)
        ebase = b * N * 8
        for k in range(4):
            cur = idxbuf[pl.ds(k * 16, 16)]
            slot = iota16 + (k * 16)
            e_k = jnp.where(slot < off, cur, first) * 8 + ebase
            for j in range(8):
                idxg[pl.ds(j * (S * NS1) + s * NS1 + k * 16, 16)] = e_k + j
        return 0

    jax.lax.fori_loop(0, S, s_body, 0)

    waits = []
    for j in range(8):
        waits.append(pltpu.async_copy(
            table_hbm.at[idxg.at[pl.ds(j * (S * NS1), S * NS1)]],
            colp.at[pl.ds(j * (S * NS1), S * NS1)], sem))
    for w in waits:
        w.wait()
    pltpu.sync_copy(colp, out_hbm.at[pl.ds(b * 8 * S * NS1, 8 * S * NS1)])


def _mlp1_kernel(g_ref,
                 w1_ref, b1_ref, g1_ref, e1_ref,
                 w2_ref, b2_ref, g2_ref, e2_ref,
                 w3_ref, b3_ref, g3_ref, e3_ref,
                 o_ref):
    x = g_ref[0]
    h = jnp.dot(w1_ref[...], x, precision=HI) + b1_ref[...]
    h = jax.nn.relu(g1_ref[...] * h + e1_ref[...])
    h = jnp.dot(w2_ref[...], h, precision=HI) + b2_ref[...]
    h = jax.nn.relu(g2_ref[...] * h + e2_ref[...])
    h = jnp.dot(w3_ref[...], h, precision=HI) + b3_ref[...]
    f = jax.nn.relu(g3_ref[...] * h + e3_ref[...])
    o_ref[0] = jnp.max(f.reshape(128, S, NS1), axis=2)


def _tail_kernel(cxl_ref, cyl_ref, czl_ref, cxs_ref, cys_ref, czs_ref,
                 f1_ref, *p_refs):
    p = [r[...] for r in p_refs[:-1]]
    o_ref = p_refs[-1]
    (w21, b21, g21, e21, w22, b22, g22, e22, w23, b23, g23, e23,
     w31, b31, g31, e31, w32, b32, g32, e32, w33, b33, g33, e33,
     wf1, bf1, gf1, ef1, wf2, bf2, gf2, ef2) = p

    xs = cxl_ref[...]
    ys = cyl_ref[...]
    zs = czl_ref[...]
    iota_n = jax.lax.broadcasted_iota(I32, (B, S), 1)

    def body(t, carry):
        dists, far, ax, ay, az = carry
        sel = iota_n == far
        cx = jnp.sum(jnp.where(sel, xs, 0.0), axis=1, keepdims=True)
        cy = jnp.sum(jnp.where(sel, ys, 0.0), axis=1, keepdims=True)
        cz = jnp.sum(jnp.where(sel, zs, 0.0), axis=1, keepdims=True)
        ax = jnp.where(iota_n == t, cx, ax)
        ay = jnp.where(iota_n == t, cy, ay)
        az = jnp.where(iota_n == t, cz, az)
        dx = xs - cx
        dy = ys - cy
        dz = zs - cz
        d = dx * dx + dy * dy + dz * dz
        dists = jnp.minimum(dists, d)
        m = jnp.max(dists, axis=1, keepdims=True)
        far = jnp.min(jnp.where(dists == m, iota_n, S), axis=1, keepdims=True)
        return dists, far, ax, ay, az

    dists0 = jnp.full((B, S), 1e10, F32)
    far0 = jnp.zeros((B, 1), I32)
    a0 = jnp.zeros((B, S), F32)
    _, _, c2x, c2y, c2z = jax.lax.fori_loop(0, S, body,
                                            (dists0, far0, a0, a0, a0))

    ddx = c2x.reshape(B, 1, S) - cxs_ref[...]
    ddy = c2y.reshape(B, 1, S) - cys_ref[...]
    ddz = c2z.reshape(B, 1, S) - czs_ref[...]
    d2 = ddx * ddx + ddy * ddy + ddz * ddz
    mask2 = d2 <= R2_2
    iota_3 = jax.lax.broadcasted_iota(I32, (B, S, S), 1)
    cnt2 = jnp.sum(jnp.where(mask2, 1.0, 0.0), axis=1, keepdims=True)
    dmin2 = jnp.min(d2, axis=1, keepdims=True)
    near2 = jnp.min(jnp.where(d2 == dmin2, iota_3, S), axis=1, keepdims=True)
    mask2f = jnp.where(mask2, 1.0, 0.0).astype(F32)
    near2f = jnp.where(iota_3 == near2, 1.0, 0.0).astype(F32)
    vf2 = jnp.where(cnt2 > 0, mask2f, near2f)

    x2 = f1_ref[...].reshape(B * S, 128)
    h = jnp.dot(x2, w21, precision=HI) + b21
    h = jax.nn.relu(g21 * h + e21)
    h = jnp.dot(h, w22, precision=HI) + b22
    h = jax.nn.relu(g22 * h + e22)
    h = jnp.dot(h, w23, precision=HI) + b23
    f2 = jax.nn.relu(g23 * h + e23)
    f2v = f2.reshape(B, S, 256)

    cols = []
    for s in range(S):
        m = vf2[:, :, s:s + 1]
        cols.append(jnp.max(jnp.where(m > 0, f2v, 0.0), axis=1, keepdims=True))
    out2 = jnp.concatenate(cols, axis=1)

    x3 = out2.reshape(B * S, 256)
    h = jnp.dot(x3, w31, precision=HI) + b31
    h = jax.nn.relu(g31 * h + e31)
    h = jnp.dot(h, w32, precision=HI) + b32
    h = jax.nn.relu(g32 * h + e32)
    h = jnp.dot(h, w33, precision=HI) + b33
    f3 = jax.nn.relu(g33 * h + e33)
    pooled = jnp.max(f3.reshape(B, S, 512), axis=1)

    y = jnp.dot(pooled, wf1, precision=HI) + bf1
    y = jax.nn.relu(gf1 * y + ef1)
    y = jnp.dot(y, wf2, precision=HI) + bf2
    y = jax.nn.relu(gf2 * y + ef2)
    o_ref[...] = y


def kernel(pc, params):
    pc = pc.astype(F32)
    xs = pc[:, :, 0]
    ys = pc[:, :, 1]
    zs = pc[:, :, 2]

    cx, cy, cz = pl.pallas_call(
        _fps_kernel,
        out_shape=[jax.ShapeDtypeStruct((B, S), F32)] * 3,
    )(xs, ys, zs)

    cxs = cx[:, :, None]
    cys = cy[:, :, None]
    czs = cz[:, :, None]
    xt = jnp.concatenate(
        [jnp.transpose(pc, (0, 2, 1)), jnp.zeros((B, 2, N), F32)], axis=1)

    vmask, ccnt, near = pl.pallas_call(
        _mask_kernel,
        grid=(B,),
        in_specs=[
            pl.BlockSpec((1, 8, N), lambda b: (b, 0, 0)),
            pl.BlockSpec((1, S, 1), lambda b: (b, 0, 0)),
            pl.BlockSpec((1, S, 1), lambda b: (b, 0, 0)),
            pl.BlockSpec((1, S, 1), lambda b: (b, 0, 0)),
        ],
        out_specs=[
            pl.BlockSpec((1, S, N), lambda b: (b, 0, 0)),
            pl.BlockSpec((1, S, NCHUNK), lambda b: (b, 0, 0)),
            pl.BlockSpec((1, S, 1), lambda b: (b, 0, 0)),
        ],
        out_shape=[
            jax.ShapeDtypeStruct((B, S, N), F32),
            jax.ShapeDtypeStruct((B, S, NCHUNK), I32),
            jax.ShapeDtypeStruct((B, S, 1), I32),
        ],
    )(xt, cxs, cys, czs)

    table = jnp.concatenate([pc, jnp.zeros((B, N, 2), F32)],
                            axis=2).reshape(B * N * 8)

    mesh = plsc.VectorSubcoreMesh(core_axis_name="c", subcore_axis_name="s")
    grouped = pl.kernel(
        _sc_gather_body,
        mesh=mesh,
        out_type=jax.ShapeDtypeStruct((B * 8 * S * NS1,), F32),
        scratch_types=[
            pltpu.VMEM((N,), F32),
            pltpu.VMEM((S * NCHUNK + 16,), I32),
            pltpu.VMEM((S + 16,), I32),
            pltpu.VMEM((256,), I32),
            pltpu.VMEM((8 * S * NS1,), I32),
            pltpu.VMEM((8 * S * NS1,), F32),
            pltpu.SemaphoreType.DMA,
        ],
    )(vmask.reshape(B * S * N), ccnt.reshape(B * S * NCHUNK),
      near.reshape(B * S), table)

    sa1 = params["sa1"]
    w_in = []
    for (w, b, g, e) in sa1:
        wt = jnp.transpose(w)
        if wt.shape[1] == 6:
            wt = jnp.concatenate([wt, jnp.zeros((wt.shape[0], 2), F32)],
                                 axis=1)
        w_in += [wt, b[:, None], g[:, None], e[:, None]]

    f1t = pl.pallas_call(
        _mlp1_kernel,
        grid=(B,),
        in_specs=[
            pl.BlockSpec((1, 8, S * NS1), lambda b: (b, 0, 0)),
        ] + [pl.BlockSpec(w.shape, lambda b: (0, 0)) for w in w_in],
        out_specs=pl.BlockSpec((1, 128, S), lambda b: (b, 0, 0)),
        out_shape=jax.ShapeDtypeStruct((B, 128, S), F32),
    )(grouped.reshape(B, 8, S * NS1), *w_in)
    f1 = jnp.transpose(f1t, (0, 2, 1))

    p_in = []
    for name in ("sa2", "sa3", "fc"):
        for (w, b, g, e) in params[name]:
            p_in += [w, b[None, :], g[None, :], e[None, :]]

    out = pl.pallas_call(
        _tail_kernel,
        out_shape=jax.ShapeDtypeStruct((B, 512), F32),
    )(cx, cy, cz, cxs, cys, czs, f1, *p_in)
    return out
